# Initial kernel scaffold; baseline (speedup 1.0000x reference)
import jax
import jax.numpy as jnp
from jax.experimental import pallas as pl


def kernel(x, b_pre, W_enc, b_enc, W_dec, b_dec):
    raise NotImplementedError("write your pallas kernel here")


# speedup vs baseline: 9.6976x; 9.6976x over previous
import functools

import jax
import jax.numpy as jnp
from jax.experimental import pallas as pl
from jax.experimental.pallas import tpu as pltpu

N_TOK_ = 4096
D_MODEL_ = 2048
D_SAE_ = 16384
K_ = 64

MINI32 = -2147483648


def _sortable(z):
    b = jax.lax.bitcast_convert_type(z, jnp.int32)
    return jnp.where(b < 0, jnp.bitwise_xor(~b, jnp.int32(MINI32)), b)


def _encode_kernel(x_ref, bpre_ref, w_ref, benc_ref, z_ref):
    xb = x_ref[...] - bpre_ref[...]
    zb = jax.lax.dot_general(
        xb.astype(jnp.bfloat16), w_ref[...].astype(jnp.bfloat16),
        dimension_numbers=(((1,), (1,)), ((), ())),
        preferred_element_type=jnp.float32,
    )
    z_ref[...] = zb + benc_ref[...]


def _select_kernel(z_ref, tau_ref, *, k):
    s = _sortable(z_ref[...])

    def body(i, p):
        bit = jax.lax.shift_left(jnp.int32(1), 31 - i)
        cand = jnp.bitwise_or(p, bit)
        candb = jnp.bitwise_xor(cand, jnp.int32(MINI32))
        cnt = jnp.sum((s >= candb).astype(jnp.int32), axis=1, keepdims=True)
        return jnp.where(cnt >= k, cand, p)

    p = jax.lax.fori_loop(0, 32, body, jnp.zeros(tau_ref.shape, jnp.int32))
    tau_ref[...] = jnp.bitwise_xor(p, jnp.int32(MINI32))


def _decode_kernel(z_ref, tau_ref, wd_ref, bdec_ref, zs_ref, rec_ref):
    zb = z_ref[...]
    mask = _sortable(zb) >= tau_ref[...]
    zs = jnp.where(mask, zb, 0.0)
    zs_ref[...] = zs

    contrib = jax.lax.dot_general(
        zs.astype(jnp.bfloat16), wd_ref[...].astype(jnp.bfloat16),
        dimension_numbers=(((1,), (1,)), ((), ())),
        preferred_element_type=jnp.float32,
    )

    @pl.when(pl.program_id(1) == 0)
    def _init():
        rec_ref[...] = jnp.broadcast_to(bdec_ref[...], rec_ref.shape)

    rec_ref[...] += contrib


@jax.jit
def kernel(x, b_pre, W_enc, b_enc, W_dec, b_dec):
    n_tok, d_model = x.shape
    d_sae = W_enc.shape[0]

    TB_E, FB_E = 512, 1024
    z = pl.pallas_call(
        _encode_kernel,
        grid=(d_sae // FB_E, n_tok // TB_E),
        in_specs=[
            pl.BlockSpec((TB_E, d_model), lambda f, t: (t, 0)),
            pl.BlockSpec((1, d_model), lambda f, t: (0, 0)),
            pl.BlockSpec((FB_E, d_model), lambda f, t: (f, 0)),
            pl.BlockSpec((1, FB_E), lambda f, t: (0, f)),
        ],
        out_specs=pl.BlockSpec((TB_E, FB_E), lambda f, t: (t, f)),
        out_shape=jax.ShapeDtypeStruct((n_tok, d_sae), jnp.float32),
        compiler_params=pltpu.CompilerParams(
            dimension_semantics=("arbitrary", "arbitrary")),
    )(x, b_pre.reshape(1, d_model), W_enc, b_enc.reshape(1, d_sae))

    TB_S = 256
    tau = pl.pallas_call(
        functools.partial(_select_kernel, k=K_),
        grid=(n_tok // TB_S,),
        in_specs=[pl.BlockSpec((TB_S, d_sae), lambda t: (t, 0))],
        out_specs=pl.BlockSpec((TB_S, 1), lambda t: (t, 0)),
        out_shape=jax.ShapeDtypeStruct((n_tok, 1), jnp.int32),
        compiler_params=pltpu.CompilerParams(
            dimension_semantics=("arbitrary",)),
    )(z)

    TB_D, KB_D = 512, 512
    z_sparse, recon = pl.pallas_call(
        _decode_kernel,
        grid=(n_tok // TB_D, d_sae // KB_D),
        in_specs=[
            pl.BlockSpec((TB_D, KB_D), lambda t, kk: (t, kk)),
            pl.BlockSpec((TB_D, 1), lambda t, kk: (t, 0)),
            pl.BlockSpec((d_model, KB_D), lambda t, kk: (0, kk)),
            pl.BlockSpec((1, d_model), lambda t, kk: (0, 0)),
        ],
        out_specs=[
            pl.BlockSpec((TB_D, KB_D), lambda t, kk: (t, kk)),
            pl.BlockSpec((TB_D, d_model), lambda t, kk: (t, 0)),
        ],
        out_shape=[
            jax.ShapeDtypeStruct((n_tok, d_sae), jnp.float32),
            jax.ShapeDtypeStruct((n_tok, d_model), jnp.float32),
        ],
        compiler_params=pltpu.CompilerParams(
            dimension_semantics=("arbitrary", "arbitrary")),
    )(z, tau, W_dec, b_dec.reshape(1, d_model))

    return (recon, z_sparse)
